# Initial kernel scaffold; baseline (speedup 1.0000x reference)
import jax
import jax.numpy as jnp
from jax.experimental import pallas as pl


def kernel(x, edge_index, W1, b1, Wh, bh, W2, b2, Wout, bout):
    raise NotImplementedError("write your pallas kernel here")


# speedup vs baseline: 14.2326x; 14.2326x over previous
import functools

import jax
import jax.numpy as jnp
from jax import lax
from jax.experimental import pallas as pl
from jax.experimental.pallas import tpu as pltpu
from jax.experimental.pallas import tpu_sc as plsc

N = 10000
E = 320000
D = 128

NC = 2
NS = 16
NW = NC * NS

CHUNK = 128
NCHUNKS = E // CHUNK
CHUNK_ITERS = -(-NCHUNKS // NW)

STRIPE = 640
STRIPE_LAST = N - (NS - 1) * STRIPE
ZB = 80


def _sc_degree_body(dst_hbm, deg_a_hbm, deg_b_hbm, deg_sh, idx_d, ones_v, zbuf):
    c = lax.axis_index("c")
    s = lax.axis_index("s")
    wid = s * NC + c

    ov = jnp.ones((16,), jnp.float32)
    zv = jnp.zeros((16,), jnp.float32)
    for j in range(CHUNK // 16):
        ones_v[pl.ds(j * 16, 16)] = ov

    @pl.when(s == 0)
    def _zero():
        def fill(i, carry):
            zbuf[pl.ds(i * 16, 16)] = zv
            return carry
        lax.fori_loop(0, N // 16, fill, 0)
        pltpu.sync_copy(zbuf, deg_sh)

    plsc.subcore_barrier()

    def chunk_body(k, carry):
        ci = wid + k * NW

        @pl.when(ci < NCHUNKS)
        def _():
            pltpu.sync_copy(dst_hbm.at[pl.ds(ci * CHUNK, CHUNK)], idx_d)
            pltpu.sync_copy(ones_v, deg_sh.at[idx_d], add=True)

        return carry

    lax.fori_loop(0, CHUNK_ITERS, chunk_body, 0)
    plsc.subcore_barrier()

    @pl.when(jnp.logical_and(s == 0, c == 0))
    def _writeback_a():
        pltpu.sync_copy(deg_sh, deg_a_hbm)

    @pl.when(jnp.logical_and(s == 0, c == 1))
    def _writeback_b():
        pltpu.sync_copy(deg_sh, deg_b_hbm)


_sc_degree = functools.partial(
    pl.kernel,
    out_type=(
        jax.ShapeDtypeStruct((N,), jnp.float32),
        jax.ShapeDtypeStruct((N,), jnp.float32),
    ),
    mesh=plsc.VectorSubcoreMesh(core_axis_name="c", subcore_axis_name="s"),
    scratch_types=[
        pltpu.VMEM_SHARED((N,), jnp.float32),
        pltpu.VMEM((CHUNK,), jnp.int32),
        pltpu.VMEM((CHUNK,), jnp.float32),
        pltpu.VMEM((N,), jnp.float32),
    ],
)(_sc_degree_body)


def _sc_scatter_body(y_hbm, src_hbm, dst_hbm, z_hbm,
                     z_sh, idx_s, idx_d, rows, zbuf, sem):
    c = lax.axis_index("c")
    s = lax.axis_index("s")
    wid = s * NC + c

    zv = jnp.zeros((16,), jnp.float32)

    def fill(i, carry):
        zbuf[i // 8, pl.ds((i % 8) * 16, 16)] = zv
        return carry

    lax.fori_loop(0, ZB * (D // 16), fill, 0)

    for j in range(STRIPE // ZB):
        @pl.when(jnp.logical_or(s < NS - 1, j < STRIPE_LAST // ZB))
        def _zero():
            pltpu.sync_copy(zbuf, z_sh.at[pl.ds(s * STRIPE + j * ZB, ZB)])

    plsc.subcore_barrier()

    def chunk_body(k, carry):
        ci = wid + k * NW

        @pl.when(ci < NCHUNKS)
        def _():
            base = ci * CHUNK
            pltpu.sync_copy(src_hbm.at[pl.ds(base, CHUNK)], idx_s)
            pltpu.sync_copy(dst_hbm.at[pl.ds(base, CHUNK)], idx_d)
            pltpu.async_copy(y_hbm.at[idx_s], rows, sem).wait()
            pltpu.sync_copy(rows, z_sh.at[idx_d], add=True)

        return carry

    lax.fori_loop(0, CHUNK_ITERS, chunk_body, 0)
    plsc.subcore_barrier()

    @pl.when(s < NS - 1)
    def _wb_full():
        pltpu.sync_copy(
            z_sh.at[pl.ds(s * STRIPE, STRIPE)],
            z_hbm.at[c, pl.ds(s * STRIPE, STRIPE)],
        )

    @pl.when(s == NS - 1)
    def _wb_last():
        pltpu.sync_copy(
            z_sh.at[pl.ds(s * STRIPE, STRIPE_LAST)],
            z_hbm.at[c, pl.ds(s * STRIPE, STRIPE_LAST)],
        )


_sc_scatter = functools.partial(
    pl.kernel,
    out_type=jax.ShapeDtypeStruct((NC, N, D), jnp.float32),
    mesh=plsc.VectorSubcoreMesh(core_axis_name="c", subcore_axis_name="s"),
    scratch_types=[
        pltpu.VMEM_SHARED((N, D), jnp.float32),
        pltpu.VMEM((CHUNK,), jnp.int32),
        pltpu.VMEM((CHUNK,), jnp.int32),
        pltpu.VMEM((CHUNK, D), jnp.float32),
        pltpu.VMEM((ZB, D), jnp.float32),
        pltpu.SemaphoreType.DMA,
    ],
)(_sc_scatter_body)


def _tc_prep_body(deg_a_ref, deg_b_ref, x_ref, w_ref, dinv_ref, y_ref):
    deg = deg_a_ref[...] + deg_b_ref[...] + 1.0
    dinv = lax.rsqrt(deg)
    dinv_ref[...] = dinv
    xw = jnp.dot(x_ref[...], w_ref[...], preferred_element_type=jnp.float32)
    y_ref[...] = xw * dinv[:, None]


def _tc_prep(deg_a, deg_b, x, w):
    return pl.pallas_call(
        _tc_prep_body,
        out_shape=(
            jax.ShapeDtypeStruct((N,), jnp.float32),
            jax.ShapeDtypeStruct((N, D), jnp.float32),
        ),
    )(deg_a, deg_b, x, w)


def _tc_mid_body(z_ref, y_ref, dinv_ref, b_ref, w_ref, out_ref):
    dinv = dinv_ref[...]
    zsum = z_ref[0] + z_ref[1] + y_ref[...]
    h = jnp.maximum(zsum * dinv[:, None] + b_ref[...], 0.0)
    hw = jnp.dot(h, w_ref[...], preferred_element_type=jnp.float32)
    out_ref[...] = hw * dinv[:, None]


def _tc_mid(z, y, dinv, b, w):
    return pl.pallas_call(
        _tc_mid_body,
        out_shape=jax.ShapeDtypeStruct((N, D), jnp.float32),
    )(z, y, dinv, b, w)


def _tc_final_body(z_ref, y_ref, dinv_ref, b_ref, wout_ref, bout_ref,
                   out_ref, h_ref):
    dinv = dinv_ref[...]
    zsum = z_ref[0] + z_ref[1] + y_ref[...]
    h = zsum * dinv[:, None] + b_ref[...]
    h_ref[...] = h
    out_ref[...] = (
        jnp.sum(h * wout_ref[...][:, 0][None, :], axis=1, keepdims=True)
        + bout_ref[...]
    )


def _tc_final(z, y, dinv, b, wout, bout):
    return pl.pallas_call(
        _tc_final_body,
        out_shape=(
            jax.ShapeDtypeStruct((N, 1), jnp.float32),
            jax.ShapeDtypeStruct((N, D), jnp.float32),
        ),
    )(z, y, dinv, b, wout, bout)


def kernel(x, edge_index, W1, b1, Wh, bh, W2, b2, Wout, bout):
    src = edge_index[0]
    dst = edge_index[1]

    deg_a, deg_b = _sc_degree(dst)
    dinv, y1 = _tc_prep(deg_a, deg_b, x, W1)

    z1 = _sc_scatter(y1, src, dst)
    y2 = _tc_mid(z1, y1, dinv, b1, Wh)

    z2 = _sc_scatter(y2, src, dst)
    y3 = _tc_mid(z2, y2, dinv, bh, W2)

    z3 = _sc_scatter(y3, src, dst)
    out, h3 = _tc_final(z3, y3, dinv, b2, Wout, bout)

    return (out, h3)
